# baseline (speedup 1.0000x reference)
import jax
import jax.numpy as jnp
from jax import lax
from jax.experimental import pallas as pl
from jax.experimental.pallas import tpu as pltpu
from jax.experimental.pallas import tpu_sc as plsc

VOCAB = 1000000
EMBED = 64
EMBED_PAD = 128
BATCH = 16384
CTX = 50

NC = 2
NS = 16
NW = NC * NS

ROWS_PER_DMA = 1
CHUNK = ROWS_PER_DMA * CTX
CHUNK_PAD = 56
RPW = BATCH // NW
CPW = RPW // ROWS_PER_DMA
GSIZE = 8
NGRP = CPW // GSIZE
GROWS = GSIZE * ROWS_PER_DMA
NLANE = EMBED // 16
INV_CTX = 1.0 / CTX


def _cbow_body(idx_hbm, table_hbm, out_hbm, idx_v, buf_v, out_v, sem0, sem1):
    wid = lax.axis_index("s") * NC + lax.axis_index("c")
    base_chunk = wid * CPW

    pltpu.sync_copy(idx_hbm.at[pl.ds(base_chunk * CHUNK_PAD, CPW * CHUNK_PAD)],
                    idx_v)

    sems = (sem0, sem1)

    def issue_group(g, parity):
        for k in range(GSIZE):
            c = g * GSIZE + k
            pltpu.make_async_copy(
                table_hbm.at[idx_v.at[pl.ds(c * CHUNK_PAD, CHUNK_PAD)]],
                buf_v.at[parity, k],
                sems[parity],
            ).start()

    def drain_group(g, parity):
        for k in range(GSIZE):
            c = g * GSIZE + k
            pltpu.make_async_copy(
                table_hbm.at[idx_v.at[pl.ds(c * CHUNK_PAD, CHUNK_PAD)]],
                buf_v.at[parity, k],
                sems[parity],
            ).wait()

    def reduce_group(g, parity):
        def row_body(rr, carry):
            c = rr // ROWS_PER_DMA
            r = rr % ROWS_PER_DMA
            j0 = r * CTX
            acc = [buf_v[parity, c, j0, pl.ds(16 * q, 16)] for q in range(NLANE)]
            for j in range(1, CTX):
                for q in range(NLANE):
                    acc[q] += buf_v[parity, c, j0 + j, pl.ds(16 * q, 16)]
            orow = g * GROWS + rr
            for q in range(NLANE):
                out_v[orow, pl.ds(16 * q, 16)] = acc[q] * INV_CTX
            return carry
        lax.fori_loop(0, GROWS, row_body, 0, unroll=False)

    issue_group(0, 0)

    def group_body(g, carry):
        parity = lax.rem(g, 2)

        @pl.when(g + 1 < NGRP)
        def _issue_next():
            nparity = lax.rem(g + 1, 2)

            @pl.when(nparity == 0)
            def _():
                issue_group(g + 1, 0)

            @pl.when(nparity == 1)
            def _():
                issue_group(g + 1, 1)

        @pl.when(parity == 0)
        def _p0():
            drain_group(g, 0)
            reduce_group(g, 0)

        @pl.when(parity == 1)
        def _p1():
            drain_group(g, 1)
            reduce_group(g, 1)

        return carry

    lax.fori_loop(0, NGRP, group_body, 0, unroll=False)

    pltpu.sync_copy(out_v, out_hbm.at[pl.ds(wid * RPW, RPW)])


@jax.jit
def _cbow(idx_padded, table):
    mesh = plsc.VectorSubcoreMesh(core_axis_name="c", subcore_axis_name="s")
    f = pl.kernel(
        _cbow_body,
        out_type=jax.ShapeDtypeStruct((BATCH, EMBED), jnp.float32),
        mesh=mesh,
        scratch_types=[
            pltpu.VMEM((CPW * CHUNK_PAD,), jnp.int32),
            pltpu.VMEM((2, GSIZE, CHUNK_PAD, EMBED), jnp.float32),
            pltpu.VMEM((RPW, EMBED), jnp.float32),
            pltpu.SemaphoreType.DMA,
            pltpu.SemaphoreType.DMA,
        ],
        compiler_params=pltpu.CompilerParams(use_tc_tiling_on_sc=False),
    )
    return f(idx_padded, table)


def kernel(inputs, table):
    idx = inputs.astype(jnp.int32).reshape(BATCH // ROWS_PER_DMA, CHUNK)
    idx = jnp.pad(idx, ((0, 0), (0, CHUNK_PAD - CHUNK))).reshape(-1)
    return _cbow(idx, table)


# speedup vs baseline: 3.4927x; 3.4927x over previous
import jax
import jax.numpy as jnp
from jax import lax
from jax.experimental import pallas as pl
from jax.experimental.pallas import tpu as pltpu
from jax.experimental.pallas import tpu_sc as plsc

VOCAB = 1000000
EMBED = 64
EMBED_PAD = 128
BATCH = 16384
CTX = 50

NC = 2
NS = 16
NW = NC * NS

ROWS_PER_DMA = 8
CHUNK = ROWS_PER_DMA * CTX
CHUNK_PAD = 400
RPW = BATCH // NW
CPW = RPW // ROWS_PER_DMA
GSIZE = 1
NGRP = CPW // GSIZE
GROWS = GSIZE * ROWS_PER_DMA
NLANE = EMBED // 16
INV_CTX = 1.0 / CTX


def _cbow_body(idx_hbm, table_hbm, out_hbm, idx_v, buf_v, out_v, sem0, sem1):
    wid = lax.axis_index("s") * NC + lax.axis_index("c")
    base_chunk = wid * CPW

    pltpu.sync_copy(idx_hbm.at[pl.ds(base_chunk * CHUNK_PAD, CPW * CHUNK_PAD)],
                    idx_v)

    sems = (sem0, sem1)

    def issue_group(g, parity):
        for k in range(GSIZE):
            c = g * GSIZE + k
            pltpu.make_async_copy(
                table_hbm.at[idx_v.at[pl.ds(c * CHUNK_PAD, CHUNK_PAD)]],
                buf_v.at[parity, k],
                sems[parity],
            ).start()

    def drain_group(g, parity):
        for k in range(GSIZE):
            c = g * GSIZE + k
            pltpu.make_async_copy(
                table_hbm.at[idx_v.at[pl.ds(c * CHUNK_PAD, CHUNK_PAD)]],
                buf_v.at[parity, k],
                sems[parity],
            ).wait()

    def reduce_group(g, parity):
        def row_body(rr, carry):
            c = rr // ROWS_PER_DMA
            r = rr % ROWS_PER_DMA
            j0 = r * CTX
            acc = [buf_v[parity, c, j0, pl.ds(16 * q, 16)] for q in range(NLANE)]
            for j in range(1, CTX):
                for q in range(NLANE):
                    acc[q] += buf_v[parity, c, j0 + j, pl.ds(16 * q, 16)]
            orow = g * GROWS + rr
            for q in range(NLANE):
                out_v[orow, pl.ds(16 * q, 16)] = acc[q] * INV_CTX
            return carry
        lax.fori_loop(0, GROWS, row_body, 0, unroll=False)

    issue_group(0, 0)

    def group_body(g, carry):
        parity = lax.rem(g, 2)

        @pl.when(g + 1 < NGRP)
        def _issue_next():
            nparity = lax.rem(g + 1, 2)

            @pl.when(nparity == 0)
            def _():
                issue_group(g + 1, 0)

            @pl.when(nparity == 1)
            def _():
                issue_group(g + 1, 1)

        @pl.when(parity == 0)
        def _p0():
            drain_group(g, 0)
            reduce_group(g, 0)

        @pl.when(parity == 1)
        def _p1():
            drain_group(g, 1)
            reduce_group(g, 1)

        return carry

    lax.fori_loop(0, NGRP, group_body, 0, unroll=False)

    pltpu.sync_copy(out_v, out_hbm.at[pl.ds(wid * RPW, RPW)])


@jax.jit
def _cbow(idx_padded, table):
    mesh = plsc.VectorSubcoreMesh(core_axis_name="c", subcore_axis_name="s")
    f = pl.kernel(
        _cbow_body,
        out_type=jax.ShapeDtypeStruct((BATCH, EMBED), jnp.float32),
        mesh=mesh,
        scratch_types=[
            pltpu.VMEM((CPW * CHUNK_PAD,), jnp.int32),
            pltpu.VMEM((2, GSIZE, CHUNK_PAD, EMBED), jnp.float32),
            pltpu.VMEM((RPW, EMBED), jnp.float32),
            pltpu.SemaphoreType.DMA,
            pltpu.SemaphoreType.DMA,
        ],
        compiler_params=pltpu.CompilerParams(use_tc_tiling_on_sc=False),
    )
    return f(idx_padded, table)


def kernel(inputs, table):
    idx = inputs.astype(jnp.int32).reshape(BATCH // ROWS_PER_DMA, CHUNK)
    idx = jnp.pad(idx, ((0, 0), (0, CHUNK_PAD - CHUNK))).reshape(-1)
    return _cbow(idx, table)
